# Initial kernel scaffold; baseline (speedup 1.0000x reference)
import jax
import jax.numpy as jnp
from jax.experimental import pallas as pl


def kernel(features, labels):
    raise NotImplementedError("write your pallas kernel here")


# speedup vs baseline: 1.9685x; 1.9685x over previous
import functools

import jax
import jax.numpy as jnp
from jax import lax
from jax.experimental import pallas as pl
from jax.experimental.pallas import tpu as pltpu
from jax.experimental.pallas import tpu_sc as plsc

NUM_CLASSES = 256
D = 512
N = 32768
EPS = 1e-08

NC = 2
NS = 16
NW = NC * NS
DH = D // NC
ROWS_PER_W = N // NS
CHUNK = 128
NCHUNKS = ROWS_PER_W // CHUNK
LBLK = 4096

_mesh = plsc.VectorSubcoreMesh(core_axis_name="c", subcore_axis_name="s")


@functools.partial(
    pl.kernel,
    out_type=jax.ShapeDtypeStruct((NW * NUM_CLASSES, DH), jnp.float32),
    mesh=_mesh,
    scratch_types=[
        pltpu.VMEM((CHUNK, DH), jnp.float32),
        pltpu.VMEM((CHUNK,), jnp.int32),
        pltpu.VMEM((NUM_CLASSES, DH), jnp.float32),
    ],
)
def _sc_segment_sum(feat_hbm, lab_hbm, psums_hbm, rows_v, lab_v, acc_v):
    c = lax.axis_index("c")
    s = lax.axis_index("s")
    wid = c * NS + s
    base = s * ROWS_PER_W
    col0 = c * DH

    zeros16 = jnp.zeros((16,), jnp.float32)

    def _zacc(i, _):
        for j in range(DH // 16):
            acc_v[i, pl.ds(j * 16, 16)] = zeros16
        return 0
    lax.fori_loop(0, NUM_CLASSES, _zacc, 0)

    def _chunk(k, _):
        r0 = base + k * CHUNK
        pltpu.sync_copy(feat_hbm.at[pl.ds(r0, CHUNK), pl.ds(col0, DH)],
                        rows_v)
        pltpu.sync_copy(lab_hbm.at[pl.ds(r0, CHUNK)], lab_v)

        def _group(g, _):
            i0 = g * 16
            lab16 = lab_v[pl.ds(i0, 16)]
            for l in range(16):
                lab = lab16[l]
                for j in range(DH // 16):
                    plsc.addupdate(acc_v.at[lab, pl.ds(j * 16, 16)],
                                   rows_v[i0 + l, pl.ds(j * 16, 16)])
            return 0
        lax.fori_loop(0, CHUNK // 16, _group, 0)
        return 0
    lax.fori_loop(0, NCHUNKS, _chunk, 0)

    pltpu.sync_copy(acc_v, psums_hbm.at[pl.ds(wid * NUM_CLASSES, NUM_CLASSES)])


def _tc_finish(psums_ref, lab_ref, out_ref):
    psums = psums_ref[...]
    suml = psums[:NUM_CLASSES]
    sumr = psums[NS * NUM_CLASSES:(NS + 1) * NUM_CLASSES]
    for w in range(1, NS):
        suml = suml + psums[w * NUM_CLASSES:(w + 1) * NUM_CLASSES]
        sumr = sumr + psums[(NS + w) * NUM_CLASSES:(NS + w + 1) * NUM_CLASSES]

    cls = lax.broadcasted_iota(jnp.int32, (NUM_CLASSES, LBLK), 0)
    counts = jnp.zeros((NUM_CLASSES,), jnp.float32)
    for b in range(N // LBLK):
        blk = lab_ref[pl.ds(b, 1), :]
        eq = (blk == cls).astype(jnp.float32)
        counts = counts + jnp.sum(eq, axis=1)

    present = counts > 0.0
    safe = jnp.maximum(counts, 1.0)
    cent_l = jnp.where(present[:, None], suml / safe[:, None], 0.0)
    cent_r = jnp.where(present[:, None], sumr / safe[:, None], 0.0)
    norms = (jnp.sum(cent_l * cent_l, axis=1)
             + jnp.sum(cent_r * cent_r, axis=1))
    dims = (((1,), (1,)), ((), ()))
    gram = (lax.dot_general(cent_l, cent_l, dims,
                            preferred_element_type=jnp.float32,
                            precision=lax.Precision.HIGHEST)
            + lax.dot_general(cent_r, cent_r, dims,
                              preferred_element_type=jnp.float32,
                              precision=lax.Precision.HIGHEST))
    dist_sq = jnp.maximum(norms[:, None] + norms[None, :] - 2.0 * gram, 0.0)
    ii = lax.broadcasted_iota(jnp.int32, (NUM_CLASSES, NUM_CLASSES), 0)
    jj = lax.broadcasted_iota(jnp.int32, (NUM_CLASSES, NUM_CLASSES), 1)
    valid = (ii < jj) & present[:, None] & present[None, :]
    safe_sq = jnp.where(valid, dist_sq, 1.0)
    distance = jnp.sqrt(safe_sq) / 16.0
    terms = jnp.where(valid, jnp.exp(-(distance + EPS)), 0.0)
    out_ref[...] = jnp.sum(terms).reshape(1, 1)


_finish = pl.pallas_call(
    _tc_finish,
    out_shape=jax.ShapeDtypeStruct((1, 1), jnp.float32),
)


def kernel(features, labels):
    labels = labels.astype(jnp.int32)
    psums = _sc_segment_sum(features, labels)
    loss = _finish(psums, labels.reshape(N // LBLK, LBLK))
    return loss.reshape(())


# speedup vs baseline: 5.0072x; 2.5436x over previous
import functools

import jax
import jax.numpy as jnp
from jax import lax
from jax.experimental import pallas as pl
from jax.experimental.pallas import tpu as pltpu
from jax.experimental.pallas import tpu_sc as plsc

NUM_CLASSES = 256
D = 512
N = 32768
EPS = 1e-08

NC = 2
NS = 16
NW = NC * NS
DH = D // NC
ROWS_PER_W = N // NS
CHUNK = 128
NCHUNKS = ROWS_PER_W // CHUNK
LBLK = 4096

_mesh = plsc.VectorSubcoreMesh(core_axis_name="c", subcore_axis_name="s")


@functools.partial(
    pl.kernel,
    out_type=jax.ShapeDtypeStruct((NW * NUM_CLASSES, DH), jnp.float32),
    mesh=_mesh,
    scratch_types=[
        pltpu.VMEM((CHUNK, DH), jnp.float32),
        pltpu.VMEM((CHUNK,), jnp.int32),
        pltpu.VMEM((NUM_CLASSES, DH), jnp.float32),
    ],
)
def _sc_segment_sum(feat_hbm, lab_hbm, psums_hbm, rows_v, lab_v, acc_v):
    c = lax.axis_index("c")
    s = lax.axis_index("s")
    wid = c * NS + s
    base = s * ROWS_PER_W
    col0 = c * DH

    zeros16 = jnp.zeros((16,), jnp.float32)

    def _zacc(i, _):
        for j in range(DH // 16):
            acc_v[i, pl.ds(j * 16, 16)] = zeros16
        return 0
    lax.fori_loop(0, NUM_CLASSES, _zacc, 0)

    def _chunk(k, _):
        r0 = base + k * CHUNK
        pltpu.sync_copy(feat_hbm.at[pl.ds(r0, CHUNK), pl.ds(col0, DH)],
                        rows_v)
        pltpu.sync_copy(lab_hbm.at[pl.ds(r0, CHUNK)], lab_v)

        @functools.partial(plsc.parallel_loop, 0, CHUNK // 16, unroll=2)
        def _group(g):
            i0 = g * 16
            lab16 = lab_v[pl.ds(i0, 16)]
            for l in range(16):
                lab = lab16[l]
                for j in range(DH // 16):
                    plsc.addupdate(acc_v.at[lab, pl.ds(j * 16, 16)],
                                   rows_v[i0 + l, pl.ds(j * 16, 16)])
        return 0
    lax.fori_loop(0, NCHUNKS, _chunk, 0)

    pltpu.sync_copy(acc_v, psums_hbm.at[pl.ds(wid * NUM_CLASSES, NUM_CLASSES)])


def _tc_finish(psums_ref, lab_ref, out_ref):
    psums = psums_ref[...]
    suml = psums[:NUM_CLASSES]
    sumr = psums[NS * NUM_CLASSES:(NS + 1) * NUM_CLASSES]
    for w in range(1, NS):
        suml = suml + psums[w * NUM_CLASSES:(w + 1) * NUM_CLASSES]
        sumr = sumr + psums[(NS + w) * NUM_CLASSES:(NS + w + 1) * NUM_CLASSES]

    cls = lax.broadcasted_iota(jnp.int32, (NUM_CLASSES, LBLK), 0)
    counts = jnp.zeros((NUM_CLASSES,), jnp.float32)
    for b in range(N // LBLK):
        blk = lab_ref[pl.ds(b, 1), :]
        eq = (blk == cls).astype(jnp.float32)
        counts = counts + jnp.sum(eq, axis=1)

    present = counts > 0.0
    safe = jnp.maximum(counts, 1.0)
    cent_l = jnp.where(present[:, None], suml / safe[:, None], 0.0)
    cent_r = jnp.where(present[:, None], sumr / safe[:, None], 0.0)
    norms = (jnp.sum(cent_l * cent_l, axis=1)
             + jnp.sum(cent_r * cent_r, axis=1))
    dims = (((1,), (1,)), ((), ()))
    gram = (lax.dot_general(cent_l, cent_l, dims,
                            preferred_element_type=jnp.float32,
                            precision=lax.Precision.HIGHEST)
            + lax.dot_general(cent_r, cent_r, dims,
                              preferred_element_type=jnp.float32,
                              precision=lax.Precision.HIGHEST))
    dist_sq = jnp.maximum(norms[:, None] + norms[None, :] - 2.0 * gram, 0.0)
    ii = lax.broadcasted_iota(jnp.int32, (NUM_CLASSES, NUM_CLASSES), 0)
    jj = lax.broadcasted_iota(jnp.int32, (NUM_CLASSES, NUM_CLASSES), 1)
    valid = (ii < jj) & present[:, None] & present[None, :]
    safe_sq = jnp.where(valid, dist_sq, 1.0)
    distance = jnp.sqrt(safe_sq) / 16.0
    terms = jnp.where(valid, jnp.exp(-(distance + EPS)), 0.0)
    out_ref[...] = jnp.sum(terms).reshape(1, 1)


_finish = pl.pallas_call(
    _tc_finish,
    out_shape=jax.ShapeDtypeStruct((1, 1), jnp.float32),
)


def kernel(features, labels):
    labels = labels.astype(jnp.int32)
    psums = _sc_segment_sum(features, labels)
    loss = _finish(psums, labels.reshape(N // LBLK, LBLK))
    return loss.reshape(())
